# baseline (speedup 1.0000x reference)
import functools

import jax
import jax.numpy as jnp
from jax import lax
from jax.experimental import pallas as pl
from jax.experimental.pallas import tpu as pltpu
from jax.experimental.pallas import tpu_sc as plsc

N_NODES = 10000
N_PAD = 10240
E_EDGES = 320000
D_FEAT = 128

NC = 2
NS = 16
NW = NC * NS
CHUNK = 80
N_HALF = 64
N_CHUNKS = 2 * N_HALF
E_PER_W = CHUNK * N_CHUNKS
E_PAD = E_PER_W * NW
ROWS_PER_TILE = N_PAD // NS


def _sc_aggregate(row3, col3, x):
  mesh = plsc.VectorSubcoreMesh(
      core_axis_name="c", subcore_axis_name="s", num_cores=NC, num_subcores=NS)

  @functools.partial(
      pl.kernel,
      mesh=mesh,
      out_type=[
          jax.ShapeDtypeStruct((NC, N_PAD, D_FEAT), jnp.float32),
          jax.ShapeDtypeStruct((NC, N_PAD), jnp.float32),
      ],
      scratch_types=[
          pltpu.VMEM((N_HALF, CHUNK), jnp.int32),
          pltpu.VMEM((N_HALF, CHUNK), jnp.int32),
          pltpu.VMEM((CHUNK, D_FEAT), jnp.float32),
          pltpu.VMEM((CHUNK, D_FEAT), jnp.float32),
          pltpu.VMEM((CHUNK,), jnp.float32),
          pltpu.VMEM_SHARED((N_PAD, D_FEAT), jnp.float32),
          pltpu.VMEM_SHARED((N_PAD,), jnp.float32),
          pltpu.SemaphoreType.DMA,
          pltpu.SemaphoreType.DMA,
          pltpu.SemaphoreType.DMA,
      ],
  )
  def agg(row_hbm, col_hbm, x_hbm, part_out, deg_out,
          row_v, col_v, buf0, buf1, ones_v, acc_sh, deg_sh,
          sem_g0, sem_g1, sem_s):
    cid = lax.axis_index("c")
    sid = lax.axis_index("s")
    wid = sid * NC + cid

    zeros16 = jnp.zeros((16,), jnp.float32)

    def zrow(i, carry):
      for j in range(D_FEAT // 16):
        buf0[i, pl.ds(j * 16, 16)] = zeros16
      return carry
    lax.fori_loop(0, CHUNK, zrow, 0)
    for j in range(CHUNK // 16):
      ones_v[pl.ds(j * 16, 16)] = jnp.ones((16,), jnp.float32)

    base = sid * ROWS_PER_TILE
    for k in range(ROWS_PER_TILE // CHUNK):
      pltpu.sync_copy(buf0, acc_sh.at[pl.ds(base + k * CHUNK, CHUNK)])
    for k in range(ROWS_PER_TILE // D_FEAT):
      pltpu.sync_copy(buf0.at[0], deg_sh.at[pl.ds(base + k * D_FEAT, D_FEAT)])

    pltpu.sync_copy(row_hbm.at[wid].at[0], row_v)
    pltpu.sync_copy(col_hbm.at[wid].at[0], col_v)

    plsc.subcore_barrier()

    def run_half():
      def pair(i, carry):
        j0 = 2 * i
        g0 = pltpu.async_copy(x_hbm.at[row_v.at[j0]], buf0, sem_g0)
        g1 = pltpu.async_copy(x_hbm.at[row_v.at[j0 + 1]], buf1, sem_g1)
        g0.wait()
        pltpu.sync_copy(buf0, acc_sh.at[col_v.at[j0]], add=True)
        pltpu.sync_copy(ones_v, deg_sh.at[col_v.at[j0]], add=True)
        g1.wait()
        pltpu.sync_copy(buf1, acc_sh.at[col_v.at[j0 + 1]], add=True)
        pltpu.sync_copy(ones_v, deg_sh.at[col_v.at[j0 + 1]], add=True)
        return carry
      lax.fori_loop(0, N_HALF // 2, pair, 0)

    run_half()
    pltpu.sync_copy(row_hbm.at[wid].at[1], row_v)
    pltpu.sync_copy(col_hbm.at[wid].at[1], col_v)
    run_half()

    plsc.subcore_barrier()

    pltpu.sync_copy(acc_sh.at[pl.ds(base, ROWS_PER_TILE)],
                    part_out.at[cid].at[pl.ds(base, ROWS_PER_TILE)])
    pltpu.sync_copy(deg_sh.at[pl.ds(base, ROWS_PER_TILE)],
                    deg_out.at[cid].at[pl.ds(base, ROWS_PER_TILE)])

  return agg(row3, col3, x)


def _tc_mlps(x, part, degp, Wf1, bf1, Wf2, bf2, Ws1, bs1, Ws2, bs2,
             Wc1a, Wc1b, bc1, Wc2, bc2):
  BLK = 400
  grid = (N_NODES // BLK,)

  def body(x_ref, p_ref, d_ref, wf1, bf1r, wf2, bf2r, ws1, bs1r, ws2, bs2r,
           wc1a, wc1b, bc1r, wc2, bc2r, o_ref):
    xb = x_ref[...]
    hf = jnp.maximum(jnp.dot(xb, wf1[...],
                             preferred_element_type=jnp.float32) + bf1r[...],
                     0.0)
    hf = jnp.dot(hf, wf2[...], preferred_element_type=jnp.float32) + bf2r[...]

    p = p_ref[0] + p_ref[1]
    d = d_ref[0, :, :] + d_ref[1, :, :]
    ax = p * (1.0 / jnp.maximum(d, 1.0))
    hs = jnp.maximum(jnp.dot(ax, ws1[...],
                             preferred_element_type=jnp.float32) + bs1r[...],
                     0.0)
    hs = jnp.dot(hs, ws2[...], preferred_element_type=jnp.float32) + bs2r[...]

    h1 = jnp.maximum(jnp.dot(hf, wc1a[...], preferred_element_type=jnp.float32)
                     + jnp.dot(hs, wc1b[...],
                               preferred_element_type=jnp.float32)
                     + bc1r[...], 0.0)
    o_ref[...] = (jnp.dot(h1, wc2[...], preferred_element_type=jnp.float32)
                  + bc2r[...])

  full = lambda shape: pl.BlockSpec(shape, lambda i: (0,) * len(shape))
  return pl.pallas_call(
      body,
      grid=grid,
      in_specs=[
          pl.BlockSpec((BLK, D_FEAT), lambda i: (i, 0)),
          pl.BlockSpec((NC, BLK, D_FEAT), lambda i: (0, i, 0)),
          pl.BlockSpec((NC, BLK, 1), lambda i: (0, i, 0)),
          full((D_FEAT, 128)), full((1, 128)),
          full((128, 128)), full((1, 128)),
          full((D_FEAT, 128)), full((1, 128)),
          full((128, 128)), full((1, 128)),
          full((128, 128)), full((128, 128)), full((1, 128)),
          full((128, 64)), full((1, 64)),
      ],
      out_specs=pl.BlockSpec((BLK, 64), lambda i: (i, 0)),
      out_shape=jax.ShapeDtypeStruct((N_NODES, 64), jnp.float32),
  )(x, part, degp, Wf1, bf1, Wf2, bf2, Ws1, bs1, Ws2, bs2,
    Wc1a, Wc1b, bc1, Wc2, bc2)


def kernel(x, edge_index, Wf1, bf1, Wf2, bf2, Ws1, bs1, Ws2, bs2,
           Wc1, bc1, Wc2, bc2):
  n_dummy = E_PAD - E_EDGES
  row_p = jnp.concatenate(
      [edge_index[0], jnp.zeros((n_dummy,), jnp.int32)])
  col_p = jnp.concatenate(
      [edge_index[1], jnp.full((n_dummy,), N_PAD - 1, jnp.int32)])
  row3 = row_p.reshape(NW, 2, N_HALF, CHUNK)
  col3 = col_p.reshape(NW, 2, N_HALF, CHUNK)

  part, degp = _sc_aggregate(row3, col3, x)

  out = _tc_mlps(
      x, part, degp.reshape(NC, N_PAD, 1),
      Wf1, bf1.reshape(1, 128), Wf2, bf2.reshape(1, 128),
      Ws1, bs1.reshape(1, 128), Ws2, bs2.reshape(1, 128),
      Wc1[:128], Wc1[128:], bc1.reshape(1, 128),
      Wc2, bc2.reshape(1, 64))
  return out


# speedup vs baseline: 2.4606x; 2.4606x over previous
import functools

import jax
import jax.numpy as jnp
from jax import lax
from jax.experimental import pallas as pl
from jax.experimental.pallas import tpu as pltpu
from jax.experimental.pallas import tpu_sc as plsc

N_NODES = 10000
N_PAD = 10240
E_EDGES = 320000
D_FEAT = 128

NC = 2
NS = 16
NW = NC * NS
CHUNK = 80
N_HALF = 64
N_CHUNKS = 2 * N_HALF
E_PER_W = CHUNK * N_CHUNKS
E_PAD = E_PER_W * NW
ROWS_PER_TILE = N_PAD // NS


def _sc_aggregate(row3, col3, x):
  mesh = plsc.VectorSubcoreMesh(
      core_axis_name="c", subcore_axis_name="s", num_cores=NC, num_subcores=NS)

  @functools.partial(
      pl.kernel,
      mesh=mesh,
      out_type=[
          jax.ShapeDtypeStruct((NC, N_PAD, D_FEAT), jnp.float32),
          jax.ShapeDtypeStruct((NC, N_PAD), jnp.float32),
      ],
      scratch_types=[
          pltpu.VMEM((N_HALF, CHUNK), jnp.int32),
          pltpu.VMEM((N_HALF, CHUNK), jnp.int32),
          pltpu.VMEM((CHUNK, D_FEAT), jnp.float32),
          pltpu.VMEM((CHUNK, D_FEAT), jnp.float32),
          pltpu.VMEM((CHUNK,), jnp.float32),
          pltpu.VMEM_SHARED((N_PAD, D_FEAT), jnp.float32),
          pltpu.VMEM_SHARED((N_PAD,), jnp.float32),
          pltpu.SemaphoreType.DMA,
          pltpu.SemaphoreType.DMA,
          pltpu.SemaphoreType.DMA,
      ],
  )
  def agg(row_hbm, col_hbm, x_hbm, part_out, deg_out,
          row_v, col_v, buf0, buf1, ones_v, acc_sh, deg_sh,
          sem_g0, sem_g1, sem_s):
    cid = lax.axis_index("c")
    sid = lax.axis_index("s")
    wid = sid * NC + cid

    zeros16 = jnp.zeros((16,), jnp.float32)

    def zrow(i, carry):
      for j in range(D_FEAT // 16):
        buf0[i, pl.ds(j * 16, 16)] = zeros16
      return carry
    lax.fori_loop(0, CHUNK, zrow, 0)
    for j in range(CHUNK // 16):
      ones_v[pl.ds(j * 16, 16)] = jnp.ones((16,), jnp.float32)

    base = sid * ROWS_PER_TILE
    for k in range(ROWS_PER_TILE // CHUNK):
      pltpu.sync_copy(buf0, acc_sh.at[pl.ds(base + k * CHUNK, CHUNK)])
    for k in range(ROWS_PER_TILE // D_FEAT):
      pltpu.sync_copy(buf0.at[0], deg_sh.at[pl.ds(base + k * D_FEAT, D_FEAT)])

    pltpu.sync_copy(row_hbm.at[wid].at[0], row_v)
    pltpu.sync_copy(col_hbm.at[wid].at[0], col_v)

    plsc.subcore_barrier()

    def run_half():
      def pair(i, carry):
        j0 = 2 * i
        g0 = pltpu.async_copy(x_hbm.at[row_v.at[j0]], buf0, sem_g0)
        g1 = pltpu.async_copy(x_hbm.at[row_v.at[j0 + 1]], buf1, sem_g1)
        g0.wait()
        pltpu.sync_copy(buf0, acc_sh.at[col_v.at[j0]], add=True)
        pltpu.sync_copy(ones_v, deg_sh.at[col_v.at[j0]], add=True)
        g1.wait()
        pltpu.sync_copy(buf1, acc_sh.at[col_v.at[j0 + 1]], add=True)
        pltpu.sync_copy(ones_v, deg_sh.at[col_v.at[j0 + 1]], add=True)
        return carry
      lax.fori_loop(0, N_HALF // 2, pair, 0)

    run_half()
    pltpu.sync_copy(row_hbm.at[wid].at[1], row_v)
    pltpu.sync_copy(col_hbm.at[wid].at[1], col_v)
    run_half()

    plsc.subcore_barrier()

    pltpu.sync_copy(acc_sh.at[pl.ds(base, ROWS_PER_TILE)],
                    part_out.at[cid].at[pl.ds(base, ROWS_PER_TILE)])
    pltpu.sync_copy(deg_sh.at[pl.ds(base, ROWS_PER_TILE)],
                    deg_out.at[cid].at[pl.ds(base, ROWS_PER_TILE)])

  return agg(row3, col3, x)


def _tc_mlps(x, part, degp, Wf1, bf1, Wf2, bf2, Ws1, bs1, Ws2, bs2,
             Wc1a, Wc1b, bc1, Wc2, bc2):
  BLK = 400
  grid = (N_NODES // BLK,)

  def body(x_ref, p_ref, d_ref, wf1, bf1r, wf2, bf2r, ws1, bs1r, ws2, bs2r,
           wc1a, wc1b, bc1r, wc2, bc2r, o_ref):
    xb = x_ref[...]
    hf = jnp.maximum(jnp.dot(xb, wf1[...],
                             preferred_element_type=jnp.float32) + bf1r[...],
                     0.0)
    hf = jnp.dot(hf, wf2[...], preferred_element_type=jnp.float32) + bf2r[...]

    p = p_ref[0] + p_ref[1]
    d = d_ref[0, :, :] + d_ref[1, :, :]
    ax = p * (1.0 / jnp.maximum(d, 1.0))
    hs = jnp.maximum(jnp.dot(ax, ws1[...],
                             preferred_element_type=jnp.float32) + bs1r[...],
                     0.0)
    hs = jnp.dot(hs, ws2[...], preferred_element_type=jnp.float32) + bs2r[...]

    h1 = jnp.maximum(jnp.dot(hf, wc1a[...], preferred_element_type=jnp.float32)
                     + jnp.dot(hs, wc1b[...],
                               preferred_element_type=jnp.float32)
                     + bc1r[...], 0.0)
    o_ref[...] = (jnp.dot(h1, wc2[...], preferred_element_type=jnp.float32)
                  + bc2r[...])

  full = lambda shape: pl.BlockSpec(shape, lambda i: (0,) * len(shape))
  return pl.pallas_call(
      body,
      grid=grid,
      in_specs=[
          pl.BlockSpec((BLK, D_FEAT), lambda i: (i, 0)),
          pl.BlockSpec((NC, BLK, D_FEAT), lambda i: (0, i, 0)),
          pl.BlockSpec((NC, BLK, 1), lambda i: (0, i, 0)),
          full((D_FEAT, 128)), full((1, 128)),
          full((128, 128)), full((1, 128)),
          full((D_FEAT, 128)), full((1, 128)),
          full((128, 128)), full((1, 128)),
          full((128, 128)), full((128, 128)), full((1, 128)),
          full((128, 64)), full((1, 64)),
      ],
      out_specs=pl.BlockSpec((BLK, 64), lambda i: (i, 0)),
      out_shape=jax.ShapeDtypeStruct((N_NODES, 64), jnp.float32),
  )(x, part, degp, Wf1, bf1, Wf2, bf2, Ws1, bs1, Ws2, bs2,
    Wc1a, Wc1b, bc1, Wc2, bc2)


def kernel(x, edge_index, Wf1, bf1, Wf2, bf2, Ws1, bs1, Ws2, bs2,
           Wc1, bc1, Wc2, bc2):
  n_dummy = E_PAD - E_EDGES
  dummy_idx = jnp.arange(n_dummy, dtype=jnp.int32)
  row_p = jnp.concatenate([edge_index[0], dummy_idx % N_NODES])
  col_p = jnp.concatenate(
      [edge_index[1], N_NODES + dummy_idx % (N_PAD - N_NODES)])
  row3 = row_p.reshape(NW, 2, N_HALF, CHUNK)
  col3 = col_p.reshape(NW, 2, N_HALF, CHUNK)

  part, degp = _sc_aggregate(row3, col3, x)

  out = _tc_mlps(
      x, part, degp.reshape(NC, N_PAD, 1),
      Wf1, bf1.reshape(1, 128), Wf2, bf2.reshape(1, 128),
      Ws1, bs1.reshape(1, 128), Ws2, bs2.reshape(1, 128),
      Wc1[:128], Wc1[128:], bc1.reshape(1, 128),
      Wc2, bc2.reshape(1, 64))
  return out


# speedup vs baseline: 2.6297x; 1.0687x over previous
import functools

import jax
import jax.numpy as jnp
from jax import lax
from jax.experimental import pallas as pl
from jax.experimental.pallas import tpu as pltpu
from jax.experimental.pallas import tpu_sc as plsc

N_NODES = 10000
N_PAD = 10240
E_EDGES = 320000
D_FEAT = 128

NC = 2
NS = 16
NW = NC * NS
CHUNK = 80
NBUF = 4
N_STAGE = 16
N_STAGES = 8
N_CHUNKS = N_STAGE * N_STAGES
E_PER_W = CHUNK * N_CHUNKS
E_PAD = E_PER_W * NW
ROWS_PER_TILE = N_PAD // NS


def _sc_aggregate(row3, col3, x):
  mesh = plsc.VectorSubcoreMesh(
      core_axis_name="c", subcore_axis_name="s", num_cores=NC, num_subcores=NS)

  @functools.partial(
      pl.kernel,
      mesh=mesh,
      out_type=[
          jax.ShapeDtypeStruct((NC, N_PAD, D_FEAT), jnp.float32),
          jax.ShapeDtypeStruct((NC, N_PAD), jnp.float32),
      ],
      scratch_types=[
          pltpu.VMEM((N_STAGE, CHUNK), jnp.int32),
          pltpu.VMEM((N_STAGE, CHUNK), jnp.int32),
          [pltpu.VMEM((CHUNK, D_FEAT), jnp.float32) for _ in range(NBUF)],
          pltpu.VMEM((CHUNK,), jnp.float32),
          pltpu.VMEM_SHARED((N_PAD, D_FEAT), jnp.float32),
          pltpu.VMEM_SHARED((N_PAD,), jnp.float32),
          [pltpu.SemaphoreType.DMA for _ in range(NBUF)],
          pltpu.SemaphoreType.DMA,
          pltpu.SemaphoreType.DMA,
      ],
  )
  def agg(row_hbm, col_hbm, x_hbm, part_out, deg_out,
          row_v, col_v, bufs, ones_v, acc_sh, deg_sh,
          sem_g, sem_s, sem_d):
    cid = lax.axis_index("c")
    sid = lax.axis_index("s")
    wid = sid * NC + cid

    zeros16 = jnp.zeros((16,), jnp.float32)

    def zrow(i, carry):
      for j in range(D_FEAT // 16):
        bufs[0][i, pl.ds(j * 16, 16)] = zeros16
      return carry
    lax.fori_loop(0, CHUNK, zrow, 0)
    for j in range(CHUNK // 16):
      ones_v[pl.ds(j * 16, 16)] = jnp.ones((16,), jnp.float32)

    base = sid * ROWS_PER_TILE
    for k in range(ROWS_PER_TILE // CHUNK):
      pltpu.sync_copy(bufs[0], acc_sh.at[pl.ds(base + k * CHUNK, CHUNK)])
    for k in range(ROWS_PER_TILE // D_FEAT):
      pltpu.sync_copy(bufs[0].at[0],
                      deg_sh.at[pl.ds(base + k * D_FEAT, D_FEAT)])

    pltpu.sync_copy(row_hbm.at[wid].at[0], row_v)
    pltpu.sync_copy(col_hbm.at[wid].at[0], col_v)

    plsc.subcore_barrier()

    def run_stage():
      def group(i, carry):
        j0 = NBUF * i
        gs = [pltpu.async_copy(x_hbm.at[row_v.at[j0 + k]], bufs[k], sem_g[k])
              for k in range(NBUF)]
        ss = []
        for k in range(NBUF):
          gs[k].wait()
          ss.append(pltpu.async_copy(bufs[k], acc_sh.at[col_v.at[j0 + k]],
                                     sem_s, add=True))
          ss.append(pltpu.async_copy(ones_v, deg_sh.at[col_v.at[j0 + k]],
                                     sem_d, add=True))
        for s in ss:
          s.wait()
        return carry
      lax.fori_loop(0, N_STAGE // NBUF, group, 0)

    run_stage()
    for h in range(1, N_STAGES):
      pltpu.sync_copy(row_hbm.at[wid].at[h], row_v)
      pltpu.sync_copy(col_hbm.at[wid].at[h], col_v)
      run_stage()

    plsc.subcore_barrier()

    pltpu.sync_copy(acc_sh.at[pl.ds(base, ROWS_PER_TILE)],
                    part_out.at[cid].at[pl.ds(base, ROWS_PER_TILE)])
    pltpu.sync_copy(deg_sh.at[pl.ds(base, ROWS_PER_TILE)],
                    deg_out.at[cid].at[pl.ds(base, ROWS_PER_TILE)])

  return agg(row3, col3, x)


def _tc_mlps(x, part, degp, Wf1, bf1, Wf2, bf2, Ws1, bs1, Ws2, bs2,
             Wc1a, Wc1b, bc1, Wc2, bc2):
  BLK = 400
  grid = (N_NODES // BLK,)

  def body(x_ref, p_ref, d_ref, wf1, bf1r, wf2, bf2r, ws1, bs1r, ws2, bs2r,
           wc1a, wc1b, bc1r, wc2, bc2r, o_ref):
    xb = x_ref[...]
    hf = jnp.maximum(jnp.dot(xb, wf1[...],
                             preferred_element_type=jnp.float32) + bf1r[...],
                     0.0)
    hf = jnp.dot(hf, wf2[...], preferred_element_type=jnp.float32) + bf2r[...]

    p = p_ref[0] + p_ref[1]
    d = d_ref[0, :, :] + d_ref[1, :, :]
    ax = p * (1.0 / jnp.maximum(d, 1.0))
    hs = jnp.maximum(jnp.dot(ax, ws1[...],
                             preferred_element_type=jnp.float32) + bs1r[...],
                     0.0)
    hs = jnp.dot(hs, ws2[...], preferred_element_type=jnp.float32) + bs2r[...]

    h1 = jnp.maximum(jnp.dot(hf, wc1a[...], preferred_element_type=jnp.float32)
                     + jnp.dot(hs, wc1b[...],
                               preferred_element_type=jnp.float32)
                     + bc1r[...], 0.0)
    o_ref[...] = (jnp.dot(h1, wc2[...], preferred_element_type=jnp.float32)
                  + bc2r[...])

  full = lambda shape: pl.BlockSpec(shape, lambda i: (0,) * len(shape))
  return pl.pallas_call(
      body,
      grid=grid,
      in_specs=[
          pl.BlockSpec((BLK, D_FEAT), lambda i: (i, 0)),
          pl.BlockSpec((NC, BLK, D_FEAT), lambda i: (0, i, 0)),
          pl.BlockSpec((NC, BLK, 1), lambda i: (0, i, 0)),
          full((D_FEAT, 128)), full((1, 128)),
          full((128, 128)), full((1, 128)),
          full((D_FEAT, 128)), full((1, 128)),
          full((128, 128)), full((1, 128)),
          full((128, 128)), full((128, 128)), full((1, 128)),
          full((128, 64)), full((1, 64)),
      ],
      out_specs=pl.BlockSpec((BLK, 64), lambda i: (i, 0)),
      out_shape=jax.ShapeDtypeStruct((N_NODES, 64), jnp.float32),
  )(x, part, degp, Wf1, bf1, Wf2, bf2, Ws1, bs1, Ws2, bs2,
    Wc1a, Wc1b, bc1, Wc2, bc2)


def kernel(x, edge_index, Wf1, bf1, Wf2, bf2, Ws1, bs1, Ws2, bs2,
           Wc1, bc1, Wc2, bc2):
  n_dummy = E_PAD - E_EDGES
  dummy_idx = jnp.arange(n_dummy, dtype=jnp.int32)
  row_p = jnp.concatenate([edge_index[0], dummy_idx % N_NODES])
  col_p = jnp.concatenate(
      [edge_index[1], N_NODES + dummy_idx % (N_PAD - N_NODES)])
  row3 = row_p.reshape(NW, N_STAGES, N_STAGE, CHUNK)
  col3 = col_p.reshape(NW, N_STAGES, N_STAGE, CHUNK)

  part, degp = _sc_aggregate(row3, col3, x)

  out = _tc_mlps(
      x, part, degp.reshape(NC, N_PAD, 1),
      Wf1, bf1.reshape(1, 128), Wf2, bf2.reshape(1, 128),
      Ws1, bs1.reshape(1, 128), Ws2, bs2.reshape(1, 128),
      Wc1[:128], Wc1[128:], bc1.reshape(1, 128),
      Wc2, bc2.reshape(1, 64))
  return out


# speedup vs baseline: 2.7914x; 1.0615x over previous
import functools

import jax
import jax.numpy as jnp
from jax import lax
from jax.experimental import pallas as pl
from jax.experimental.pallas import tpu as pltpu
from jax.experimental.pallas import tpu_sc as plsc

N_NODES = 10000
N_PAD = 10240
E_EDGES = 320000
D_FEAT = 128

NC = 2
NS = 16
NW = NC * NS
CHUNK = 80
NBUF = 4
N_STAGE = 16
N_STAGES = 8
N_CHUNKS = N_STAGE * N_STAGES
E_PER_W = CHUNK * N_CHUNKS
E_PAD = E_PER_W * NW
ROWS_PER_TILE = N_PAD // NS


def _sc_aggregate(row3, col3, x):
  mesh = plsc.VectorSubcoreMesh(
      core_axis_name="c", subcore_axis_name="s", num_cores=NC, num_subcores=NS)

  @functools.partial(
      pl.kernel,
      mesh=mesh,
      out_type=[
          jax.ShapeDtypeStruct((NC, N_PAD, D_FEAT), jnp.float32),
          jax.ShapeDtypeStruct((NC, N_PAD), jnp.float32),
      ],
      scratch_types=[
          pltpu.VMEM((N_STAGE, CHUNK), jnp.int32),
          pltpu.VMEM((N_STAGE, CHUNK), jnp.int32),
          [pltpu.VMEM((CHUNK, D_FEAT), jnp.float32) for _ in range(NBUF)],
          pltpu.VMEM((CHUNK,), jnp.float32),
          pltpu.VMEM_SHARED((N_PAD, D_FEAT), jnp.float32),
          pltpu.VMEM_SHARED((N_PAD,), jnp.float32),
          [pltpu.SemaphoreType.DMA for _ in range(NBUF)],
          pltpu.SemaphoreType.DMA,
          pltpu.SemaphoreType.DMA,
      ],
  )
  def agg(row_hbm, col_hbm, x_hbm, part_out, deg_out,
          row_v, col_v, bufs, ones_v, acc_sh, deg_sh,
          sem_g, sem_s, sem_d):
    cid = lax.axis_index("c")
    sid = lax.axis_index("s")
    wid = sid * NC + cid

    zeros16 = jnp.zeros((16,), jnp.float32)

    def zrow(i, carry):
      for j in range(D_FEAT // 16):
        bufs[0][i, pl.ds(j * 16, 16)] = zeros16
      return carry
    lax.fori_loop(0, CHUNK, zrow, 0)
    for j in range(CHUNK // 16):
      ones_v[pl.ds(j * 16, 16)] = jnp.ones((16,), jnp.float32)

    base = sid * ROWS_PER_TILE
    for k in range(ROWS_PER_TILE // CHUNK):
      pltpu.sync_copy(bufs[0], acc_sh.at[pl.ds(base + k * CHUNK, CHUNK)])
    for k in range(ROWS_PER_TILE // D_FEAT):
      pltpu.sync_copy(bufs[0].at[0],
                      deg_sh.at[pl.ds(base + k * D_FEAT, D_FEAT)])

    pltpu.sync_copy(row_hbm.at[wid].at[0], row_v)
    pltpu.sync_copy(col_hbm.at[wid].at[0], col_v)

    plsc.subcore_barrier()

    def run_stage():
      def group(i, carry):
        j0 = NBUF * i
        gs = [pltpu.async_copy(x_hbm.at[row_v.at[j0 + k]], bufs[k], sem_g[k])
              for k in range(NBUF)]
        ss = []
        for k in range(NBUF):
          gs[k].wait()
          ss.append(pltpu.async_copy(bufs[k], acc_sh.at[col_v.at[j0 + k]],
                                     sem_s, add=True))
          ss.append(pltpu.async_copy(ones_v, deg_sh.at[col_v.at[j0 + k]],
                                     sem_d, add=True))
        for s in ss:
          s.wait()
        return carry
      lax.fori_loop(0, N_STAGE // NBUF, group, 0)

    run_stage()
    for h in range(1, N_STAGES):
      pltpu.sync_copy(row_hbm.at[wid].at[h], row_v)
      pltpu.sync_copy(col_hbm.at[wid].at[h], col_v)
      run_stage()

    plsc.subcore_barrier()

    pltpu.sync_copy(acc_sh.at[pl.ds(base, ROWS_PER_TILE)],
                    part_out.at[cid].at[pl.ds(base, ROWS_PER_TILE)])
    pltpu.sync_copy(deg_sh.at[pl.ds(base, ROWS_PER_TILE)],
                    deg_out.at[cid].at[pl.ds(base, ROWS_PER_TILE)])

  return agg(row3, col3, x)


_BLK = 2000
_full = lambda shape: pl.BlockSpec(shape, lambda i: (0,) * len(shape))


def _tc_feat(x, Wf1, bf1, Wf2, bf2):
  def body(x_ref, wf1, bf1r, wf2, bf2r, o_ref):
    hf = jnp.maximum(jnp.dot(x_ref[...], wf1[...],
                             preferred_element_type=jnp.float32) + bf1r[...],
                     0.0)
    o_ref[...] = (jnp.dot(hf, wf2[...], preferred_element_type=jnp.float32)
                  + bf2r[...])

  return pl.pallas_call(
      body,
      grid=(N_NODES // _BLK,),
      in_specs=[
          pl.BlockSpec((_BLK, D_FEAT), lambda i: (i, 0)),
          _full((D_FEAT, 128)), _full((1, 128)),
          _full((128, 128)), _full((1, 128)),
      ],
      out_specs=pl.BlockSpec((_BLK, 128), lambda i: (i, 0)),
      out_shape=jax.ShapeDtypeStruct((N_NODES, 128), jnp.float32),
  )(x, Wf1, bf1, Wf2, bf2)


def _tc_combine(hf, part, degp, Ws1, bs1, Ws2, bs2, Wc1a, Wc1b, bc1, Wc2, bc2):
  def body(hf_ref, p_ref, d_ref, ws1, bs1r, ws2, bs2r,
           wc1a, wc1b, bc1r, wc2, bc2r, o_ref):
    p = p_ref[0] + p_ref[1]
    d = d_ref[0, :, :] + d_ref[1, :, :]
    ax = p * (1.0 / jnp.maximum(d, 1.0))
    hs = jnp.maximum(jnp.dot(ax, ws1[...],
                             preferred_element_type=jnp.float32) + bs1r[...],
                     0.0)
    hs = jnp.dot(hs, ws2[...], preferred_element_type=jnp.float32) + bs2r[...]

    h1 = jnp.maximum(jnp.dot(hf_ref[...], wc1a[...],
                             preferred_element_type=jnp.float32)
                     + jnp.dot(hs, wc1b[...],
                               preferred_element_type=jnp.float32)
                     + bc1r[...], 0.0)
    o_ref[...] = (jnp.dot(h1, wc2[...], preferred_element_type=jnp.float32)
                  + bc2r[...])

  return pl.pallas_call(
      body,
      grid=(N_NODES // _BLK,),
      in_specs=[
          pl.BlockSpec((_BLK, 128), lambda i: (i, 0)),
          pl.BlockSpec((NC, _BLK, D_FEAT), lambda i: (0, i, 0)),
          pl.BlockSpec((NC, _BLK, 1), lambda i: (0, i, 0)),
          _full((D_FEAT, 128)), _full((1, 128)),
          _full((128, 128)), _full((1, 128)),
          _full((128, 128)), _full((128, 128)), _full((1, 128)),
          _full((128, 64)), _full((1, 64)),
      ],
      out_specs=pl.BlockSpec((_BLK, 64), lambda i: (i, 0)),
      out_shape=jax.ShapeDtypeStruct((N_NODES, 64), jnp.float32),
  )(hf, part, degp, Ws1, bs1, Ws2, bs2, Wc1a, Wc1b, bc1, Wc2, bc2)


def kernel(x, edge_index, Wf1, bf1, Wf2, bf2, Ws1, bs1, Ws2, bs2,
           Wc1, bc1, Wc2, bc2):
  n_dummy = E_PAD - E_EDGES
  dummy_idx = jnp.arange(n_dummy, dtype=jnp.int32)
  row_p = jnp.concatenate([edge_index[0], dummy_idx % N_NODES])
  col_p = jnp.concatenate(
      [edge_index[1], N_NODES + dummy_idx % (N_PAD - N_NODES)])
  row3 = row_p.reshape(NW, N_STAGES, N_STAGE, CHUNK)
  col3 = col_p.reshape(NW, N_STAGES, N_STAGE, CHUNK)

  hf = _tc_feat(x, Wf1, bf1.reshape(1, 128), Wf2, bf2.reshape(1, 128))
  part, degp = _sc_aggregate(row3, col3, x)

  out = _tc_combine(
      hf, part, degp.reshape(NC, N_PAD, 1),
      Ws1, bs1.reshape(1, 128), Ws2, bs2.reshape(1, 128),
      Wc1[:128], Wc1[128:], bc1.reshape(1, 128),
      Wc2, bc2.reshape(1, 64))
  return out
